# Initial kernel scaffold; baseline (speedup 1.0000x reference)
import functools

import jax
import jax.numpy as jnp
from jax import lax
from jax.experimental import pallas as pl
from jax.experimental.pallas import tpu as pltpu
from jax.experimental.pallas import tpu_sc as plsc

N = 10000
E = 320000
D = 128
DH = 64
NC = 2
NS = 16
NW = NC * NS
EPW = E // NW
B = 80
NB = EPW // B
NPAD = 10240
SLICE = NPAD // NS
L = 16

_mesh = plsc.VectorSubcoreMesh(core_axis_name="c", subcore_axis_name="s")


def _newton_rsqrt(d):
    ib = lax.bitcast_convert_type(d, jnp.int32)
    ib = jnp.int32(0x5F3759DF) - lax.shift_right_logical(ib, 1)
    y = lax.bitcast_convert_type(ib, jnp.float32)
    for _ in range(3):
        y = y * (1.5 - 0.5 * d * y * y)
    return y


def _splat(v, i):
    idx = jnp.full((L, 1), i, jnp.int32)
    dn = lax.GatherDimensionNumbers(
        offset_dims=(), collapsed_slice_dims=(0,), start_index_map=(0,))
    return lax.gather(v, idx, dn, (1,),
                      mode=lax.GatherScatterMode.PROMISE_IN_BOUNDS)


@functools.partial(
    pl.kernel,
    out_type=jax.ShapeDtypeStruct((NW, NPAD), jnp.float32),
    mesh=_mesh,
    scratch_types=[
        pltpu.VMEM((NB, B), jnp.int32),
        pltpu.VMEM((NB, B), jnp.int32),
        pltpu.VMEM((NB, B), jnp.float32),
        pltpu.VMEM((NPAD,), jnp.float32),
    ],
)
def _deg_kernel(row_h, col_h, ew_h, out_h, row_v, col_v, ew_v, deg_v):
    cid = lax.axis_index("c")
    sid = lax.axis_index("s")
    wid = cid * NS + sid
    pltpu.sync_copy(row_h.at[wid], row_v)
    pltpu.sync_copy(col_h.at[wid], col_v)
    pltpu.sync_copy(ew_h.at[wid], ew_v)

    def zero_body(i, c):
        deg_v[pl.ds(i * L, L)] = jnp.zeros((L,), jnp.float32)
        return c
    lax.fori_loop(0, NPAD // L, zero_body, 0)

    def edge_body(it, c):
        for g in range(B // L):
            r16 = row_v[it, pl.ds(g * L, L)]
            c16 = col_v[it, pl.ds(g * L, L)]
            w16 = ew_v[it, pl.ds(g * L, L)]
            weff = jnp.where(r16 == c16, 0.0, w16)
            plsc.addupdate_scatter(deg_v, [r16], weff)
        return c
    lax.fori_loop(0, NB, edge_body, 0)
    pltpu.sync_copy(deg_v, out_h.at[wid])


@functools.partial(
    pl.kernel,
    out_type=jax.ShapeDtypeStruct((NC, N, D), jnp.float32),
    mesh=_mesh,
    scratch_types=[
        pltpu.VMEM((NW, SLICE), jnp.float32),
        pltpu.VMEM((NPAD,), jnp.float32),
        pltpu.VMEM((NB, B), jnp.int32),
        pltpu.VMEM((NB, B), jnp.int32),
        pltpu.VMEM((NB, B), jnp.float32),
        pltpu.VMEM((B,), jnp.int32),
        pltpu.VMEM((B, D), jnp.float32),
        pltpu.VMEM_SHARED((NPAD,), jnp.float32),
        pltpu.VMEM_SHARED((NPAD, D), jnp.float32),
        pltpu.SemaphoreType.DMA,
    ],
)
def _spmm_kernel(row_h, col_h, ew_h, parts_h, x_h, y_h,
                 red_v, dis_v, row_v, col_v, ew_v, cidx_v, xr,
                 dis_sh, y_sh, sem):
    cid = lax.axis_index("c")
    sid = lax.axis_index("s")
    wid = cid * NS + sid

    pltpu.sync_copy(row_h.at[wid], row_v)
    pltpu.sync_copy(col_h.at[wid], col_v)
    pltpu.sync_copy(ew_h.at[wid], ew_v)

    pltpu.sync_copy(parts_h.at[:, pl.ds(sid * SLICE, SLICE)], red_v)

    def red_body(g, c):
        acc = jnp.zeros((L,), jnp.float32)
        for r in range(NW):
            acc = acc + red_v[r, pl.ds(g * L, L)]
        d = jnp.maximum(acc, 1e-12)
        y = _newton_rsqrt(d)
        dis_v[pl.ds(sid * SLICE + g * L, L)] = jnp.where(acc > 0.0, y, 0.0)
        return c
    lax.fori_loop(0, SLICE // L, red_body, 0)
    pltpu.sync_copy(dis_v.at[pl.ds(sid * SLICE, SLICE)],
                    dis_sh.at[pl.ds(sid * SLICE, SLICE)])
    plsc.subcore_barrier()
    pltpu.sync_copy(dis_sh, dis_v)

    def zxr_body(j, c):
        for cc in range(D // L):
            xr[j, pl.ds(cc * L, L)] = jnp.zeros((L,), jnp.float32)
        return c
    lax.fori_loop(0, B, zxr_body, 0)
    for k2 in range(SLICE // B):
        pltpu.sync_copy(xr, y_sh.at[pl.ds(sid * SLICE + k2 * B, B)])
    plsc.subcore_barrier()

    def batch_body(it, c):
        pltpu.async_copy(x_h.at[row_v.at[it]], xr, sem).wait()
        for g in range(B // L):
            r16 = row_v[it, pl.ds(g * L, L)]
            c16 = col_v[it, pl.ds(g * L, L)]
            w16 = ew_v[it, pl.ds(g * L, L)]
            weff = jnp.where(r16 == c16, 0.0, w16)
            n16 = -(plsc.load_gather(dis_v, [r16]) * weff
                    * plsc.load_gather(dis_v, [c16]))
            cidx_v[pl.ds(g * L, L)] = c16

            def scale_row(i, cc2):
                s = _splat(n16, i)
                j = g * L + i
                for cc in range(D // L):
                    xr[j, pl.ds(cc * L, L)] = xr[j, pl.ds(cc * L, L)] * s
                return cc2
            lax.fori_loop(0, L, scale_row, 0)
        pltpu.sync_copy(xr, y_sh.at[cidx_v], add=True)
        return c
    lax.fori_loop(0, NB, batch_body, 0)
    plsc.subcore_barrier()

    nlast = N - (NS - 1) * SLICE

    @pl.when(sid < NS - 1)
    def _():
        pltpu.sync_copy(y_sh.at[pl.ds(sid * SLICE, SLICE)],
                        y_h.at[cid, pl.ds(sid * SLICE, SLICE)])

    @pl.when(sid == NS - 1)
    def _():
        pltpu.sync_copy(y_sh.at[pl.ds((NS - 1) * SLICE, nlast)],
                        y_h.at[cid, pl.ds((NS - 1) * SLICE, nlast)])


def _dense_body(x_ref, y_ref, w0_ref, w1_ref, ci_ref, ct_ref, co_ref,
                wco_ref, wo_ref, bo_ref, o_ref):
    xs = x_ref[...]
    ys = y_ref[0] + y_ref[1]
    t = (jnp.dot(xs, w0_ref[...], preferred_element_type=jnp.float32)
         + jnp.dot(ys, w1_ref[...], preferred_element_type=jnp.float32))
    gi = jax.nn.sigmoid(t[:, 0:DH] + ci_ref[...])
    gt = jnp.tanh(t[:, DH:2 * DH] + ct_ref[...])
    cc = gi * gt
    go = jax.nn.sigmoid(t[:, 2 * DH:3 * DH] + co_ref[...]
                        + wco_ref[...] * cc)
    hh = go * jnp.tanh(cc)
    o_ref[...] = (jnp.dot(hh, wo_ref[...], preferred_element_type=jnp.float32)
                  + bo_ref[...])


_RB = 1000


def _dense(x, y2, w0c, w1c, ci, ct, co, wco, wo, bo):
    grid = (N // _RB,)
    return pl.pallas_call(
        _dense_body,
        grid=grid,
        in_specs=[
            pl.BlockSpec((_RB, D), lambda i: (i, 0)),
            pl.BlockSpec((NC, _RB, D), lambda i: (0, i, 0)),
            pl.BlockSpec((D, 3 * DH), lambda i: (0, 0)),
            pl.BlockSpec((D, 3 * DH), lambda i: (0, 0)),
            pl.BlockSpec((1, DH), lambda i: (0, 0)),
            pl.BlockSpec((1, DH), lambda i: (0, 0)),
            pl.BlockSpec((1, DH), lambda i: (0, 0)),
            pl.BlockSpec((1, DH), lambda i: (0, 0)),
            pl.BlockSpec((DH, 1), lambda i: (0, 0)),
            pl.BlockSpec((1, 1), lambda i: (0, 0)),
        ],
        out_specs=pl.BlockSpec((_RB, 1), lambda i: (i, 0)),
        out_shape=jax.ShapeDtypeStruct((N, 1), jnp.float32),
    )(x, y2, w0c, w1c, ci, ct, co, wco, wo, bo)


def kernel(x, edge_index, edge_weight, Wx0, Wx1, bx, Wh0, Wh1, bh, wc, b,
           W_out, b_out):
    row = edge_index[0].reshape(NW, NB, B)
    col = edge_index[1].reshape(NW, NB, B)
    ew = edge_weight.reshape(NW, NB, B)

    parts = _deg_kernel(row, col, ew)
    y2 = _spmm_kernel(row, col, ew, parts, x)

    w0c = jnp.concatenate([Wx0[0], Wx0[2], Wx0[3]], axis=1)
    w1c = jnp.concatenate([Wx1[0], Wx1[2], Wx1[3]], axis=1)
    ci = (bx[0] + bh[0] + b[0]).reshape(1, DH)
    ct = (bx[2] + bh[2] + b[2]).reshape(1, DH)
    co = (bx[3] + bh[3] + b[3]).reshape(1, DH)
    wco = wc[2].reshape(1, DH)
    bo = b_out.reshape(1, 1)

    out = _dense(x, y2, w0c, w1c, ci, ct, co, wco, W_out, bo)
    return out[:, 0]


# speedup vs baseline: 22.6559x; 22.6559x over previous
import functools

import jax
import jax.numpy as jnp
from jax import lax
from jax.experimental import pallas as pl
from jax.experimental.pallas import tpu as pltpu
from jax.experimental.pallas import tpu_sc as plsc

N = 10000
E = 320000
D = 128
DH = 64
NC = 2
NS = 16
NW = NC * NS
EPW = E // NW
B = 80
EC = 2000
NCH = EPW // EC
BPC = EC // B
NPAD = 10240
SLICE = NPAD // NS
L = 16
RC = 128

_mesh = plsc.VectorSubcoreMesh(core_axis_name="c", subcore_axis_name="s")
_sc_params = pltpu.CompilerParams(needs_layout_passes=False)


def _newton_rsqrt(d):
    ib = lax.bitcast_convert_type(d, jnp.int32)
    ib = jnp.int32(0x5F3759DF) - lax.shift_right_logical(ib, 1)
    y = lax.bitcast_convert_type(ib, jnp.float32)
    for _ in range(3):
        y = y * (1.5 - 0.5 * d * y * y)
    return y


def _splat(v, i):
    idx = jnp.full((L, 1), i, jnp.int32)
    dn = lax.GatherDimensionNumbers(
        offset_dims=(), collapsed_slice_dims=(0,), start_index_map=(0,))
    return lax.gather(v, idx, dn, (1,),
                      mode=lax.GatherScatterMode.PROMISE_IN_BOUNDS)


@functools.partial(
    pl.kernel,
    out_type=jax.ShapeDtypeStruct((NW, NPAD), jnp.float32),
    mesh=_mesh,
    scratch_types=[
        pltpu.VMEM((EPW,), jnp.int32),
        pltpu.VMEM((EPW,), jnp.int32),
        pltpu.VMEM((EPW,), jnp.float32),
        pltpu.VMEM((NPAD,), jnp.float32),
    ],
    compiler_params=_sc_params,
)
def _deg_kernel(row_h, col_h, ew_h, out_h, row_v, col_v, ew_v, deg_v):
    cid = lax.axis_index("c")
    sid = lax.axis_index("s")
    wid = cid * NS + sid
    base = wid * EPW
    pltpu.sync_copy(row_h.at[pl.ds(base, EPW)], row_v)
    pltpu.sync_copy(col_h.at[pl.ds(base, EPW)], col_v)
    pltpu.sync_copy(ew_h.at[pl.ds(base, EPW)], ew_v)

    def zero_body(i, c):
        deg_v[pl.ds(i * L, L)] = jnp.zeros((L,), jnp.float32)
        return c
    lax.fori_loop(0, NPAD // L, zero_body, 0)

    def edge_body(it, c):
        r16 = row_v[pl.ds(it * L, L)]
        c16 = col_v[pl.ds(it * L, L)]
        w16 = ew_v[pl.ds(it * L, L)]
        weff = jnp.where(r16 == c16, 0.0, w16)
        plsc.addupdate_scatter(deg_v, [r16], weff)
        return c
    lax.fori_loop(0, EPW // L, edge_body, 0)
    pltpu.sync_copy(deg_v, out_h.at[wid])


@functools.partial(
    pl.kernel,
    out_type=jax.ShapeDtypeStruct((NC, N, D), jnp.float32),
    mesh=_mesh,
    scratch_types=[
        pltpu.VMEM((NW, RC), jnp.float32),
        pltpu.VMEM((NPAD,), jnp.float32),
        pltpu.VMEM((EC,), jnp.int32),
        pltpu.VMEM((EC,), jnp.int32),
        pltpu.VMEM((EC,), jnp.float32),
        pltpu.VMEM((B,), jnp.int32),
        pltpu.VMEM((B, D), jnp.float32),
        pltpu.VMEM_SHARED((NPAD,), jnp.float32),
        pltpu.VMEM_SHARED((NPAD, D), jnp.float32),
        pltpu.SemaphoreType.DMA,
    ],
    compiler_params=_sc_params,
)
def _spmm_kernel(row_h, col_h, ew_h, parts_h, x_h, y_h,
                 red_v, dis_v, row_v, col_v, ew_v, cidx_v, xr,
                 dis_sh, y_sh, sem):
    cid = lax.axis_index("c")
    sid = lax.axis_index("s")
    wid = cid * NS + sid

    def red_chunk(q, c):
        col0 = sid * SLICE + q * RC
        pltpu.sync_copy(parts_h.at[:, pl.ds(col0, RC)], red_v)

        def red_body(g, c2):
            acc = jnp.zeros((L,), jnp.float32)
            for r in range(NW):
                acc = acc + red_v[r, pl.ds(g * L, L)]
            d = jnp.maximum(acc, 1e-12)
            y = _newton_rsqrt(d)
            dis_v[pl.ds(col0 + g * L, L)] = jnp.where(acc > 0.0, y, 0.0)
            return c2
        return lax.fori_loop(0, RC // L, red_body, c)
    lax.fori_loop(0, SLICE // RC, red_chunk, 0)
    pltpu.sync_copy(dis_v.at[pl.ds(sid * SLICE, SLICE)],
                    dis_sh.at[pl.ds(sid * SLICE, SLICE)])
    plsc.subcore_barrier()
    pltpu.sync_copy(dis_sh, dis_v)

    def zxr_body(j, c):
        for cc in range(D // L):
            xr[j, pl.ds(cc * L, L)] = jnp.zeros((L,), jnp.float32)
        return c
    lax.fori_loop(0, B, zxr_body, 0)
    for k2 in range(SLICE // B):
        pltpu.sync_copy(xr, y_sh.at[pl.ds(sid * SLICE + k2 * B, B)])
    plsc.subcore_barrier()

    def chunk_body(ch, c):
        ebase = wid * EPW + ch * EC
        pltpu.sync_copy(row_h.at[pl.ds(ebase, EC)], row_v)
        pltpu.sync_copy(col_h.at[pl.ds(ebase, EC)], col_v)
        pltpu.sync_copy(ew_h.at[pl.ds(ebase, EC)], ew_v)

        def batch_body(bt, c2):
            pltpu.async_copy(x_h.at[row_v.at[pl.ds(bt * B, B)]], xr,
                             sem).wait()
            for g in range(B // L):
                r16 = row_v[pl.ds(bt * B + g * L, L)]
                c16 = col_v[pl.ds(bt * B + g * L, L)]
                w16 = ew_v[pl.ds(bt * B + g * L, L)]
                weff = jnp.where(r16 == c16, 0.0, w16)
                n16 = -(plsc.load_gather(dis_v, [r16]) * weff
                        * plsc.load_gather(dis_v, [c16]))
                cidx_v[pl.ds(g * L, L)] = c16

                def scale_row(i, cc2):
                    s = _splat(n16, i)
                    j = g * L + i
                    for cc in range(D // L):
                        xr[j, pl.ds(cc * L, L)] = xr[j, pl.ds(cc * L, L)] * s
                    return cc2
                lax.fori_loop(0, L, scale_row, 0)
            pltpu.sync_copy(xr, y_sh.at[cidx_v], add=True)
            return c2
        return lax.fori_loop(0, BPC, batch_body, c)
    lax.fori_loop(0, NCH, chunk_body, 0)
    plsc.subcore_barrier()

    nlast = N - (NS - 1) * SLICE

    @pl.when(sid < NS - 1)
    def _():
        pltpu.sync_copy(y_sh.at[pl.ds(sid * SLICE, SLICE)],
                        y_h.at[cid, pl.ds(sid * SLICE, SLICE)])

    @pl.when(sid == NS - 1)
    def _():
        pltpu.sync_copy(y_sh.at[pl.ds((NS - 1) * SLICE, nlast)],
                        y_h.at[cid, pl.ds((NS - 1) * SLICE, nlast)])


def _dense_body(x_ref, y_ref, w0_ref, w1_ref, ci_ref, ct_ref, co_ref,
                wco_ref, wo_ref, bo_ref, o_ref):
    xs = x_ref[...]
    ys = y_ref[0] + y_ref[1]
    t = (jnp.dot(xs, w0_ref[...], preferred_element_type=jnp.float32)
         + jnp.dot(ys, w1_ref[...], preferred_element_type=jnp.float32))
    gi = jax.nn.sigmoid(t[:, 0:DH] + ci_ref[...])
    gt = jnp.tanh(t[:, DH:2 * DH] + ct_ref[...])
    cc = gi * gt
    go = jax.nn.sigmoid(t[:, 2 * DH:3 * DH] + co_ref[...]
                        + wco_ref[...] * cc)
    hh = go * jnp.tanh(cc)
    o_ref[...] = (jnp.dot(hh, wo_ref[...], preferred_element_type=jnp.float32)
                  + bo_ref[...])


_RB = 1000


def _dense(x, y2, w0c, w1c, ci, ct, co, wco, wo, bo):
    grid = (N // _RB,)
    return pl.pallas_call(
        _dense_body,
        grid=grid,
        in_specs=[
            pl.BlockSpec((_RB, D), lambda i: (i, 0)),
            pl.BlockSpec((NC, _RB, D), lambda i: (0, i, 0)),
            pl.BlockSpec((D, 3 * DH), lambda i: (0, 0)),
            pl.BlockSpec((D, 3 * DH), lambda i: (0, 0)),
            pl.BlockSpec((1, DH), lambda i: (0, 0)),
            pl.BlockSpec((1, DH), lambda i: (0, 0)),
            pl.BlockSpec((1, DH), lambda i: (0, 0)),
            pl.BlockSpec((1, DH), lambda i: (0, 0)),
            pl.BlockSpec((DH, 1), lambda i: (0, 0)),
            pl.BlockSpec((1, 1), lambda i: (0, 0)),
        ],
        out_specs=pl.BlockSpec((_RB, 1), lambda i: (i, 0)),
        out_shape=jax.ShapeDtypeStruct((N, 1), jnp.float32),
    )(x, y2, w0c, w1c, ci, ct, co, wco, wo, bo)


def kernel(x, edge_index, edge_weight, Wx0, Wx1, bx, Wh0, Wh1, bh, wc, b,
           W_out, b_out):
    row = edge_index[0]
    col = edge_index[1]

    parts = _deg_kernel(row, col, edge_weight)
    y2 = _spmm_kernel(row, col, edge_weight, parts, x)

    w0c = jnp.concatenate([Wx0[0], Wx0[2], Wx0[3]], axis=1)
    w1c = jnp.concatenate([Wx1[0], Wx1[2], Wx1[3]], axis=1)
    ci = (bx[0] + bh[0] + b[0]).reshape(1, DH)
    ct = (bx[2] + bh[2] + b[2]).reshape(1, DH)
    co = (bx[3] + bh[3] + b[3]).reshape(1, DH)
    wco = wc[2].reshape(1, DH)
    bo = b_out.reshape(1, 1)

    out = _dense(x, y2, w0c, w1c, ci, ct, co, wco, W_out, bo)
    return out[:, 0]


# speedup vs baseline: 25.9717x; 1.1464x over previous
import functools

import jax
import jax.numpy as jnp
from jax import lax
from jax.experimental import pallas as pl
from jax.experimental.pallas import tpu as pltpu
from jax.experimental.pallas import tpu_sc as plsc

N = 10000
E = 320000
D = 128
DH = 64
NC = 2
NS = 16
NW = NC * NS
EPW = E // NW
B = 80
EC = 2000
NCH = EPW // EC
BPC = EC // B
NBT = EPW // B
NPAD = 10240
SLICE = NPAD // NS
L = 16
RC = 128

_mesh = plsc.VectorSubcoreMesh(core_axis_name="c", subcore_axis_name="s")
_sc_params = pltpu.CompilerParams(needs_layout_passes=False)


def _newton_rsqrt(d):
    ib = lax.bitcast_convert_type(d, jnp.int32)
    ib = jnp.int32(0x5F3759DF) - lax.shift_right_logical(ib, 1)
    y = lax.bitcast_convert_type(ib, jnp.float32)
    for _ in range(3):
        y = y * (1.5 - 0.5 * d * y * y)
    return y


def _splat(v, i):
    idx = jnp.full((L, 1), i, jnp.int32)
    dn = lax.GatherDimensionNumbers(
        offset_dims=(), collapsed_slice_dims=(0,), start_index_map=(0,))
    return lax.gather(v, idx, dn, (1,),
                      mode=lax.GatherScatterMode.PROMISE_IN_BOUNDS)


@functools.partial(
    pl.kernel,
    out_type=jax.ShapeDtypeStruct((NW, NPAD), jnp.float32),
    mesh=_mesh,
    scratch_types=[
        pltpu.VMEM((EPW,), jnp.int32),
        pltpu.VMEM((EPW,), jnp.int32),
        pltpu.VMEM((EPW,), jnp.float32),
        pltpu.VMEM((NPAD,), jnp.float32),
    ],
    compiler_params=_sc_params,
)
def _deg_kernel(row_h, col_h, ew_h, out_h, row_v, col_v, ew_v, deg_v):
    cid = lax.axis_index("c")
    sid = lax.axis_index("s")
    wid = cid * NS + sid
    base = wid * EPW
    pltpu.sync_copy(row_h.at[pl.ds(base, EPW)], row_v)
    pltpu.sync_copy(col_h.at[pl.ds(base, EPW)], col_v)
    pltpu.sync_copy(ew_h.at[pl.ds(base, EPW)], ew_v)

    def zero_body(i, c):
        deg_v[pl.ds(i * L, L)] = jnp.zeros((L,), jnp.float32)
        return c
    lax.fori_loop(0, NPAD // L, zero_body, 0)

    def edge_body(it, c):
        r16 = row_v[pl.ds(it * L, L)]
        c16 = col_v[pl.ds(it * L, L)]
        w16 = ew_v[pl.ds(it * L, L)]
        weff = jnp.where(r16 == c16, 0.0, w16)
        plsc.addupdate_scatter(deg_v, [r16], weff)
        return c
    lax.fori_loop(0, EPW // L, edge_body, 0)
    pltpu.sync_copy(deg_v, out_h.at[wid])


@functools.partial(
    pl.kernel,
    out_type=jax.ShapeDtypeStruct((NC, N, D), jnp.float32),
    mesh=_mesh,
    scratch_types=[
        pltpu.VMEM((NW, RC), jnp.float32),
        pltpu.VMEM((NPAD,), jnp.float32),
        pltpu.VMEM((3, B), jnp.int32),
        pltpu.VMEM((3, B), jnp.int32),
        pltpu.VMEM((3, B), jnp.int32),
        pltpu.VMEM((B, D), jnp.float32),
        pltpu.VMEM((B, D), jnp.float32),
        pltpu.VMEM((B, D), jnp.float32),
        pltpu.VMEM_SHARED((NPAD,), jnp.float32),
        pltpu.VMEM_SHARED((NPAD, D), jnp.float32),
        pltpu.SemaphoreType.DMA,
        pltpu.SemaphoreType.DMA,
        pltpu.SemaphoreType.DMA,
        pltpu.SemaphoreType.DMA,
        pltpu.SemaphoreType.DMA,
        pltpu.SemaphoreType.DMA,
    ],
    compiler_params=_sc_params,
)
def _spmm_kernel(pk_h, parts_h, x_h, y_h,
                 red_v, dis_v, pk0, pk1, pk2, xr0, xr1, xr2,
                 dis_sh, y_sh, g0, g1, g2, s0, s1, s2):
    cid = lax.axis_index("c")
    sid = lax.axis_index("s")
    wid = cid * NS + sid
    tb = wid * NBT

    pk = [pk0, pk1, pk2]
    xr = [xr0, xr1, xr2]
    gs = [g0, g1, g2]
    ss = [s0, s1, s2]

    def red_chunk(q, c):
        col0 = sid * SLICE + q * RC
        pltpu.sync_copy(parts_h.at[:, pl.ds(col0, RC)], red_v)

        def red_body(g, c2):
            acc = jnp.zeros((L,), jnp.float32)
            for r in range(NW):
                acc = acc + red_v[r, pl.ds(g * L, L)]
            d = jnp.maximum(acc, 1e-12)
            y = _newton_rsqrt(d)
            dis_v[pl.ds(col0 + g * L, L)] = jnp.where(acc > 0.0, y, 0.0)
            return c2
        return lax.fori_loop(0, RC // L, red_body, c)
    lax.fori_loop(0, SLICE // RC, red_chunk, 0)
    pltpu.sync_copy(dis_v.at[pl.ds(sid * SLICE, SLICE)],
                    dis_sh.at[pl.ds(sid * SLICE, SLICE)])
    plsc.subcore_barrier()
    pltpu.sync_copy(dis_sh, dis_v)

    def zxr_body(j, c):
        for cc in range(D // L):
            xr0[j, pl.ds(cc * L, L)] = jnp.zeros((L,), jnp.float32)
        return c
    lax.fori_loop(0, B, zxr_body, 0)
    for k2 in range(SLICE // B):
        pltpu.sync_copy(xr0, y_sh.at[pl.ds(sid * SLICE + k2 * B, B)])
    plsc.subcore_barrier()

    def wait_gather(u):
        pltpu.make_async_copy(x_h.at[pk[u].at[0]], xr[u], gs[u]).wait()

    def wait_scatter(u):
        pltpu.make_async_copy(xr[u], y_sh.at[pk[u].at[1]], ss[u]).wait()

    def issue_gather(u, q):
        pltpu.sync_copy(pk_h.at[tb + q], pk[u])
        pltpu.async_copy(x_h.at[pk[u].at[0]], xr[u], gs[u])

    def scale(u):
        def grp(g, c):
            r16 = pk[u][0, pl.ds(g * L, L)]
            c16 = pk[u][1, pl.ds(g * L, L)]
            w16 = plsc.bitcast(pk[u][2, pl.ds(g * L, L)], jnp.float32)
            weff = jnp.where(r16 == c16, 0.0, w16)
            n16 = -(plsc.load_gather(dis_v, [r16]) * weff
                    * plsc.load_gather(dis_v, [c16]))
            for i in range(L):
                s = _splat(n16, i)
                j = g * L + i
                for cc in range(D // L):
                    xr[u][j, pl.ds(cc * L, L)] = (
                        xr[u][j, pl.ds(cc * L, L)] * s)
            return c
        lax.fori_loop(0, B // L, grp, 0)

    def issue_scatter(u):
        pltpu.async_copy(xr[u], y_sh.at[pk[u].at[1]], ss[u], add=True)

    def slot(u, b, prefetch):
        wait_gather(u)
        if prefetch:
            v = (u + 1) % 3

            @pl.when(b >= 2)
            def _():
                wait_scatter(v)
            issue_gather(v, b + 1)
        scale(u)
        issue_scatter(u)

    issue_gather(0, 0)

    def pipe_body(jj, c):
        b0 = 3 * jj
        slot(0, b0, True)
        slot(1, b0 + 1, True)
        slot(2, b0 + 2, True)
        return c
    lax.fori_loop(0, (NBT - 2) // 3, pipe_body, 0)
    slot(0, NBT - 2, True)
    slot(1, NBT - 1, False)
    wait_scatter(2)
    wait_scatter(0)
    wait_scatter(1)
    plsc.subcore_barrier()

    nlast = N - (NS - 1) * SLICE

    @pl.when(sid < NS - 1)
    def _():
        pltpu.sync_copy(y_sh.at[pl.ds(sid * SLICE, SLICE)],
                        y_h.at[cid, pl.ds(sid * SLICE, SLICE)])

    @pl.when(sid == NS - 1)
    def _():
        pltpu.sync_copy(y_sh.at[pl.ds((NS - 1) * SLICE, nlast)],
                        y_h.at[cid, pl.ds((NS - 1) * SLICE, nlast)])


def _dense_body(x_ref, y_ref, w0_ref, w1_ref, ci_ref, ct_ref, co_ref,
                wco_ref, wo_ref, bo_ref, o_ref):
    xs = x_ref[...]
    ys = y_ref[0] + y_ref[1]
    t = (jnp.dot(xs, w0_ref[...], preferred_element_type=jnp.float32)
         + jnp.dot(ys, w1_ref[...], preferred_element_type=jnp.float32))
    gi = jax.nn.sigmoid(t[:, 0:DH] + ci_ref[...])
    gt = jnp.tanh(t[:, DH:2 * DH] + ct_ref[...])
    cc = gi * gt
    go = jax.nn.sigmoid(t[:, 2 * DH:3 * DH] + co_ref[...]
                        + wco_ref[...] * cc)
    hh = go * jnp.tanh(cc)
    o_ref[...] = (jnp.dot(hh, wo_ref[...], preferred_element_type=jnp.float32)
                  + bo_ref[...])


_RB = 1000


def _dense(x, y2, w0c, w1c, ci, ct, co, wco, wo, bo):
    grid = (N // _RB,)
    return pl.pallas_call(
        _dense_body,
        grid=grid,
        in_specs=[
            pl.BlockSpec((_RB, D), lambda i: (i, 0)),
            pl.BlockSpec((NC, _RB, D), lambda i: (0, i, 0)),
            pl.BlockSpec((D, 3 * DH), lambda i: (0, 0)),
            pl.BlockSpec((D, 3 * DH), lambda i: (0, 0)),
            pl.BlockSpec((1, DH), lambda i: (0, 0)),
            pl.BlockSpec((1, DH), lambda i: (0, 0)),
            pl.BlockSpec((1, DH), lambda i: (0, 0)),
            pl.BlockSpec((1, DH), lambda i: (0, 0)),
            pl.BlockSpec((DH, 1), lambda i: (0, 0)),
            pl.BlockSpec((1, 1), lambda i: (0, 0)),
        ],
        out_specs=pl.BlockSpec((_RB, 1), lambda i: (i, 0)),
        out_shape=jax.ShapeDtypeStruct((N, 1), jnp.float32),
    )(x, y2, w0c, w1c, ci, ct, co, wco, wo, bo)


def kernel(x, edge_index, edge_weight, Wx0, Wx1, bx, Wh0, Wh1, bh, wc, b,
           W_out, b_out):
    row = edge_index[0]
    col = edge_index[1]

    parts = _deg_kernel(row, col, edge_weight)
    ewi = lax.bitcast_convert_type(edge_weight, jnp.int32)
    pk = jnp.stack([row.reshape(E // B, B), col.reshape(E // B, B),
                    ewi.reshape(E // B, B)], axis=1)
    y2 = _spmm_kernel(pk, parts, x)

    w0c = jnp.concatenate([Wx0[0], Wx0[2], Wx0[3]], axis=1)
    w1c = jnp.concatenate([Wx1[0], Wx1[2], Wx1[3]], axis=1)
    ci = (bx[0] + bh[0] + b[0]).reshape(1, DH)
    ct = (bx[2] + bh[2] + b[2]).reshape(1, DH)
    co = (bx[3] + bh[3] + b[3]).reshape(1, DH)
    wco = wc[2].reshape(1, DH)
    bo = b_out.reshape(1, 1)

    out = _dense(x, y2, w0c, w1c, ci, ct, co, wco, W_out, bo)
    return out[:, 0]


# speedup vs baseline: 32.8198x; 1.2637x over previous
import functools

import jax
import jax.numpy as jnp
from jax import lax
from jax.experimental import pallas as pl
from jax.experimental.pallas import tpu as pltpu
from jax.experimental.pallas import tpu_sc as plsc

N = 10000
E = 320000
D = 128
DH = 64
NC = 2
NS = 16
NW = NC * NS
EPW = E // NW
B = 80
EC = 2000
NCH = EPW // EC
BPC = EC // B
NBT = EPW // B
NPAD = 10240
SLICE = NPAD // NS
L = 16
RC = 128

_mesh = plsc.VectorSubcoreMesh(core_axis_name="c", subcore_axis_name="s")
_sc_params = pltpu.CompilerParams(needs_layout_passes=False)


def _newton_rsqrt(d):
    ib = lax.bitcast_convert_type(d, jnp.int32)
    ib = jnp.int32(0x5F3759DF) - lax.shift_right_logical(ib, 1)
    y = lax.bitcast_convert_type(ib, jnp.float32)
    for _ in range(3):
        y = y * (1.5 - 0.5 * d * y * y)
    return y


def _splat(v, i):
    idx = jnp.full((L, 1), i, jnp.int32)
    dn = lax.GatherDimensionNumbers(
        offset_dims=(), collapsed_slice_dims=(0,), start_index_map=(0,))
    return lax.gather(v, idx, dn, (1,),
                      mode=lax.GatherScatterMode.PROMISE_IN_BOUNDS)


@functools.partial(
    pl.kernel,
    out_type=jax.ShapeDtypeStruct((NW, NPAD), jnp.float32),
    mesh=_mesh,
    scratch_types=[
        pltpu.VMEM((EPW,), jnp.int32),
        pltpu.VMEM((EPW,), jnp.int32),
        pltpu.VMEM((EPW,), jnp.float32),
        pltpu.VMEM((NPAD,), jnp.float32),
    ],
    compiler_params=_sc_params,
)
def _deg_kernel(row_h, col_h, ew_h, out_h, row_v, col_v, ew_v, deg_v):
    cid = lax.axis_index("c")
    sid = lax.axis_index("s")
    wid = cid * NS + sid
    base = wid * EPW
    pltpu.sync_copy(row_h.at[pl.ds(base, EPW)], row_v)
    pltpu.sync_copy(col_h.at[pl.ds(base, EPW)], col_v)
    pltpu.sync_copy(ew_h.at[pl.ds(base, EPW)], ew_v)

    def zero_body(i, c):
        deg_v[pl.ds(i * L, L)] = jnp.zeros((L,), jnp.float32)
        return c
    lax.fori_loop(0, NPAD // L, zero_body, 0)

    def edge_body(it, c):
        r16 = row_v[pl.ds(it * L, L)]
        c16 = col_v[pl.ds(it * L, L)]
        w16 = ew_v[pl.ds(it * L, L)]
        weff = jnp.where(r16 == c16, 0.0, w16)
        plsc.addupdate_scatter(deg_v, [r16], weff)
        return c
    lax.fori_loop(0, EPW // L, edge_body, 0)
    pltpu.sync_copy(deg_v, out_h.at[wid])


@functools.partial(
    pl.kernel,
    out_type=jax.ShapeDtypeStruct((NC, N, D), jnp.float32),
    mesh=_mesh,
    scratch_types=[
        pltpu.VMEM((NW, RC), jnp.float32),
        pltpu.VMEM((NPAD,), jnp.float32),
        pltpu.VMEM((3, B), jnp.int32),
        pltpu.VMEM((3, B), jnp.int32),
        pltpu.VMEM((3, B), jnp.int32),
        pltpu.VMEM((B, D), jnp.float32),
        pltpu.VMEM((B, D), jnp.float32),
        pltpu.VMEM((B, D), jnp.float32),
        pltpu.VMEM((B,), jnp.int32),
        pltpu.VMEM((B,), jnp.int32),
        pltpu.VMEM((B,), jnp.int32),
        pltpu.VMEM_SHARED((NPAD,), jnp.float32),
        pltpu.VMEM_SHARED((NPAD, D), jnp.float32),
        pltpu.SemaphoreType.DMA,
        pltpu.SemaphoreType.DMA,
        pltpu.SemaphoreType.DMA,
        pltpu.SemaphoreType.DMA,
        pltpu.SemaphoreType.DMA,
        pltpu.SemaphoreType.DMA,
        pltpu.SemaphoreType.DMA,
        pltpu.SemaphoreType.DMA,
        pltpu.SemaphoreType.DMA,
    ],
    compiler_params=_sc_params,
)
def _spmm_kernel(pk_h, parts_h, x_h, y_h,
                 red_v, dis_v, pk0, pk1, pk2, xr0, xr1, xr2,
                 ci0, ci1, ci2, dis_sh, y_sh,
                 g0, g1, g2, s0, s1, s2, k0, k1, k2):
    cid = lax.axis_index("c")
    sid = lax.axis_index("s")
    wid = cid * NS + sid
    tb = wid * NBT

    pk = [pk0, pk1, pk2]
    xr = [xr0, xr1, xr2]
    ci = [ci0, ci1, ci2]
    gs = [g0, g1, g2]
    ss = [s0, s1, s2]
    ks = [k0, k1, k2]

    def red_chunk(q, c):
        col0 = sid * SLICE + q * RC
        pltpu.sync_copy(parts_h.at[:, pl.ds(col0, RC)], red_v)

        def red_body(g, c2):
            acc = jnp.zeros((L,), jnp.float32)
            for r in range(NW):
                acc = acc + red_v[r, pl.ds(g * L, L)]
            d = jnp.maximum(acc, 1e-12)
            y = _newton_rsqrt(d)
            dis_v[pl.ds(col0 + g * L, L)] = jnp.where(acc > 0.0, y, 0.0)
            return c2
        return lax.fori_loop(0, RC // L, red_body, c)
    lax.fori_loop(0, SLICE // RC, red_chunk, 0)
    pltpu.sync_copy(dis_v.at[pl.ds(sid * SLICE, SLICE)],
                    dis_sh.at[pl.ds(sid * SLICE, SLICE)])
    plsc.subcore_barrier()
    pltpu.sync_copy(dis_sh, dis_v)

    def zxr_body(j, c):
        for cc in range(D // L):
            xr0[j, pl.ds(cc * L, L)] = jnp.zeros((L,), jnp.float32)
        return c
    lax.fori_loop(0, B, zxr_body, 0)
    for k2 in range(SLICE // B):
        pltpu.sync_copy(xr0, y_sh.at[pl.ds(sid * SLICE + k2 * B, B)])
    plsc.subcore_barrier()

    def wait_gather(u):
        pltpu.make_async_copy(x_h.at[pk[u].at[0]], xr[u], gs[u]).wait()

    def wait_scatter(u):
        pltpu.make_async_copy(xr[u], y_sh.at[ci[u]], ss[u]).wait()

    def issue_pk(u, q):
        pltpu.async_copy(pk_h.at[tb + q], pk[u], ks[u])

    def wait_pk(u):
        pltpu.make_async_copy(pk_h.at[tb], pk[u], ks[u]).wait()

    def issue_gather(u):
        pltpu.async_copy(x_h.at[pk[u].at[0]], xr[u], gs[u])

    def scale(u):
        def grp(g, c):
            r16 = pk[u][0, pl.ds(g * L, L)]
            c16 = pk[u][1, pl.ds(g * L, L)]
            w16 = plsc.bitcast(pk[u][2, pl.ds(g * L, L)], jnp.float32)
            weff = jnp.where(r16 == c16, 0.0, w16)
            n16 = -(plsc.load_gather(dis_v, [r16]) * weff
                    * plsc.load_gather(dis_v, [c16]))
            ci[u][pl.ds(g * L, L)] = c16
            for i in range(L):
                s = _splat(n16, i)
                j = g * L + i
                for cc in range(D // L):
                    xr[u][j, pl.ds(cc * L, L)] = (
                        xr[u][j, pl.ds(cc * L, L)] * s)
            return c
        lax.fori_loop(0, B // L, grp, 0)

    def issue_scatter(u):
        pltpu.async_copy(xr[u], y_sh.at[ci[u]], ss[u], add=True)

    def slot(u, b, prefetch):
        wait_gather(u)
        if prefetch:
            v = (u + 1) % 3

            @pl.when(b >= 2)
            def _():
                wait_scatter(v)
            wait_pk(v)
            issue_gather(v)
        scale(u)
        issue_scatter(u)
        if prefetch:
            @pl.when(b + 3 < NBT)
            def _():
                issue_pk(u, b + 3)

    issue_pk(0, 0)
    issue_pk(1, 1)
    issue_pk(2, 2)
    wait_pk(0)
    issue_gather(0)

    def pipe_body(jj, c):
        b0 = 3 * jj
        slot(0, b0, True)
        slot(1, b0 + 1, True)
        slot(2, b0 + 2, True)
        return c
    lax.fori_loop(0, (NBT - 2) // 3, pipe_body, 0)
    slot(0, NBT - 2, True)
    slot(1, NBT - 1, False)
    wait_scatter(2)
    wait_scatter(0)
    wait_scatter(1)
    plsc.subcore_barrier()

    nlast = N - (NS - 1) * SLICE

    @pl.when(sid < NS - 1)
    def _():
        pltpu.sync_copy(y_sh.at[pl.ds(sid * SLICE, SLICE)],
                        y_h.at[cid, pl.ds(sid * SLICE, SLICE)])

    @pl.when(sid == NS - 1)
    def _():
        pltpu.sync_copy(y_sh.at[pl.ds((NS - 1) * SLICE, nlast)],
                        y_h.at[cid, pl.ds((NS - 1) * SLICE, nlast)])


def _dense_body(x_ref, y_ref, w0_ref, w1_ref, ci_ref, ct_ref, co_ref,
                wco_ref, wo_ref, bo_ref, o_ref):
    xs = x_ref[...]
    ys = y_ref[0] + y_ref[1]
    t = (jnp.dot(xs, w0_ref[...], preferred_element_type=jnp.float32)
         + jnp.dot(ys, w1_ref[...], preferred_element_type=jnp.float32))
    gi = jax.nn.sigmoid(t[:, 0:DH] + ci_ref[...])
    gt = jnp.tanh(t[:, DH:2 * DH] + ct_ref[...])
    cc = gi * gt
    go = jax.nn.sigmoid(t[:, 2 * DH:3 * DH] + co_ref[...]
                        + wco_ref[...] * cc)
    hh = go * jnp.tanh(cc)
    o_ref[...] = (jnp.dot(hh, wo_ref[...], preferred_element_type=jnp.float32)
                  + bo_ref[...])


_RB = 1000


def _dense(x, y2, w0c, w1c, ci, ct, co, wco, wo, bo):
    grid = (N // _RB,)
    return pl.pallas_call(
        _dense_body,
        grid=grid,
        in_specs=[
            pl.BlockSpec((_RB, D), lambda i: (i, 0)),
            pl.BlockSpec((NC, _RB, D), lambda i: (0, i, 0)),
            pl.BlockSpec((D, 3 * DH), lambda i: (0, 0)),
            pl.BlockSpec((D, 3 * DH), lambda i: (0, 0)),
            pl.BlockSpec((1, DH), lambda i: (0, 0)),
            pl.BlockSpec((1, DH), lambda i: (0, 0)),
            pl.BlockSpec((1, DH), lambda i: (0, 0)),
            pl.BlockSpec((1, DH), lambda i: (0, 0)),
            pl.BlockSpec((DH, 1), lambda i: (0, 0)),
            pl.BlockSpec((1, 1), lambda i: (0, 0)),
        ],
        out_specs=pl.BlockSpec((_RB, 1), lambda i: (i, 0)),
        out_shape=jax.ShapeDtypeStruct((N, 1), jnp.float32),
    )(x, y2, w0c, w1c, ci, ct, co, wco, wo, bo)


def kernel(x, edge_index, edge_weight, Wx0, Wx1, bx, Wh0, Wh1, bh, wc, b,
           W_out, b_out):
    row = edge_index[0]
    col = edge_index[1]

    parts = _deg_kernel(row, col, edge_weight)
    ewi = lax.bitcast_convert_type(edge_weight, jnp.int32)
    pk = jnp.stack([row.reshape(E // B, B), col.reshape(E // B, B),
                    ewi.reshape(E // B, B)], axis=1)
    y2 = _spmm_kernel(pk, parts, x)

    w0c = jnp.concatenate([Wx0[0], Wx0[2], Wx0[3]], axis=1)
    w1c = jnp.concatenate([Wx1[0], Wx1[2], Wx1[3]], axis=1)
    ci = (bx[0] + bh[0] + b[0]).reshape(1, DH)
    ct = (bx[2] + bh[2] + b[2]).reshape(1, DH)
    co = (bx[3] + bh[3] + b[3]).reshape(1, DH)
    wco = wc[2].reshape(1, DH)
    bo = b_out.reshape(1, 1)

    out = _dense(x, y2, w0c, w1c, ci, ct, co, wco, W_out, bo)
    return out[:, 0]


# speedup vs baseline: 38.0497x; 1.1594x over previous
import functools

import jax
import jax.numpy as jnp
from jax import lax
from jax.experimental import pallas as pl
from jax.experimental.pallas import tpu as pltpu
from jax.experimental.pallas import tpu_sc as plsc

N = 10000
E = 320000
D = 128
DH = 64
NC = 2
NS = 16
NW = NC * NS
EPW = E // NW
B = 80
EC = 2000
NCH = EPW // EC
BPC = EC // B
NBT = EPW // B
NPAD = 10240
SLICE = NPAD // NS
L = 16
RC = 128

_mesh = plsc.VectorSubcoreMesh(core_axis_name="c", subcore_axis_name="s")
_sc_params = pltpu.CompilerParams(needs_layout_passes=False)


def _newton_rsqrt(d):
    ib = lax.bitcast_convert_type(d, jnp.int32)
    ib = jnp.int32(0x5F3759DF) - lax.shift_right_logical(ib, 1)
    y = lax.bitcast_convert_type(ib, jnp.float32)
    for _ in range(3):
        y = y * (1.5 - 0.5 * d * y * y)
    return y


def _splat(v, i):
    idx = jnp.full((L, 1), i, jnp.int32)
    dn = lax.GatherDimensionNumbers(
        offset_dims=(), collapsed_slice_dims=(0,), start_index_map=(0,))
    return lax.gather(v, idx, dn, (1,),
                      mode=lax.GatherScatterMode.PROMISE_IN_BOUNDS)


@functools.partial(
    pl.kernel,
    out_type=jax.ShapeDtypeStruct((NW, NPAD), jnp.float32),
    mesh=_mesh,
    scratch_types=[
        pltpu.VMEM((EPW,), jnp.int32),
        pltpu.VMEM((EPW,), jnp.int32),
        pltpu.VMEM((EPW,), jnp.float32),
        pltpu.VMEM((NPAD,), jnp.float32),
    ],
    compiler_params=_sc_params,
)
def _deg_kernel(row_h, col_h, ew_h, out_h, row_v, col_v, ew_v, deg_v):
    cid = lax.axis_index("c")
    sid = lax.axis_index("s")
    wid = cid * NS + sid
    base = wid * EPW
    pltpu.sync_copy(row_h.at[pl.ds(base, EPW)], row_v)
    pltpu.sync_copy(col_h.at[pl.ds(base, EPW)], col_v)
    pltpu.sync_copy(ew_h.at[pl.ds(base, EPW)], ew_v)

    def zero_body(i, c):
        deg_v[pl.ds(i * L, L)] = jnp.zeros((L,), jnp.float32)
        return c
    lax.fori_loop(0, NPAD // L, zero_body, 0)

    def edge_body(it, c):
        r16 = row_v[pl.ds(it * L, L)]
        c16 = col_v[pl.ds(it * L, L)]
        w16 = ew_v[pl.ds(it * L, L)]
        weff = jnp.where(r16 == c16, 0.0, w16)
        plsc.addupdate_scatter(deg_v, [r16], weff)
        return c
    lax.fori_loop(0, EPW // L, edge_body, 0)
    pltpu.sync_copy(deg_v, out_h.at[wid])


@functools.partial(
    pl.kernel,
    out_type=jax.ShapeDtypeStruct((NC, N, D), jnp.float32),
    mesh=_mesh,
    scratch_types=[
        pltpu.VMEM((NW, RC), jnp.float32),
        pltpu.VMEM((NPAD,), jnp.float32),
        pltpu.VMEM((3, B), jnp.int32),
        pltpu.VMEM((3, B), jnp.int32),
        pltpu.VMEM((3, B), jnp.int32),
        pltpu.VMEM((B, D), jnp.float32),
        pltpu.VMEM((B, D), jnp.float32),
        pltpu.VMEM((B, D), jnp.float32),
        pltpu.VMEM((B,), jnp.int32),
        pltpu.VMEM((B,), jnp.int32),
        pltpu.VMEM((B,), jnp.int32),
        pltpu.VMEM_SHARED((NPAD,), jnp.float32),
        pltpu.VMEM_SHARED((NPAD, D), jnp.float32),
        pltpu.SemaphoreType.DMA,
        pltpu.SemaphoreType.DMA,
        pltpu.SemaphoreType.DMA,
        pltpu.SemaphoreType.DMA,
        pltpu.SemaphoreType.DMA,
        pltpu.SemaphoreType.DMA,
        pltpu.SemaphoreType.DMA,
        pltpu.SemaphoreType.DMA,
        pltpu.SemaphoreType.DMA,
    ],
    compiler_params=_sc_params,
)
def _spmm_kernel(pk_h, parts_h, x_h, y_h,
                 red_v, dis_v, pk0, pk1, pk2, xr0, xr1, xr2,
                 ci0, ci1, ci2, dis_sh, y_sh,
                 g0, g1, g2, s0, s1, s2, k0, k1, k2):
    cid = lax.axis_index("c")
    sid = lax.axis_index("s")
    wid = cid * NS + sid
    tb = wid * NBT

    pk = [pk0, pk1, pk2]
    xr = [xr0, xr1, xr2]
    ci = [ci0, ci1, ci2]
    gs = [g0, g1, g2]
    ss = [s0, s1, s2]
    ks = [k0, k1, k2]

    def red_chunk(q, c):
        col0 = sid * SLICE + q * RC
        pltpu.sync_copy(parts_h.at[:, pl.ds(col0, RC)], red_v)

        def red_body(g, c2):
            acc = jnp.zeros((L,), jnp.float32)
            for r in range(NW):
                acc = acc + red_v[r, pl.ds(g * L, L)]
            d = jnp.maximum(acc, 1e-12)
            y = _newton_rsqrt(d)
            dis_v[pl.ds(col0 + g * L, L)] = jnp.where(acc > 0.0, y, 0.0)
            return c2
        return lax.fori_loop(0, RC // L, red_body, c)
    lax.fori_loop(0, SLICE // RC, red_chunk, 0)
    pltpu.sync_copy(dis_v.at[pl.ds(sid * SLICE, SLICE)],
                    dis_sh.at[pl.ds(sid * SLICE, SLICE)])
    plsc.subcore_barrier()
    pltpu.sync_copy(dis_sh, dis_v)

    def zxr_body(j, c):
        for cc in range(D // L):
            xr0[j, pl.ds(cc * L, L)] = jnp.zeros((L,), jnp.float32)
        return c
    lax.fori_loop(0, B, zxr_body, 0)
    for k2 in range(SLICE // B):
        pltpu.sync_copy(xr0, y_sh.at[pl.ds(sid * SLICE + k2 * B, B)])
    plsc.subcore_barrier()

    def wait_gather(u):
        pltpu.make_async_copy(x_h.at[pk[u].at[0]], xr[u], gs[u]).wait()

    def wait_scatter(u):
        pltpu.make_async_copy(xr[u], y_sh.at[ci[u]], ss[u]).wait()

    def issue_pk(u, q):
        pltpu.async_copy(pk_h.at[tb + q], pk[u], ks[u])

    def wait_pk(u):
        pltpu.make_async_copy(pk_h.at[tb], pk[u], ks[u]).wait()

    def issue_gather(u):
        pltpu.async_copy(x_h.at[pk[u].at[0]], xr[u], gs[u])

    def scale(u):
        def grp(g, c):
            r16 = pk[u][0, pl.ds(g * L, L)]
            c16 = pk[u][1, pl.ds(g * L, L)]
            w16 = plsc.bitcast(pk[u][2, pl.ds(g * L, L)], jnp.float32)
            weff = jnp.where(r16 == c16, 0.0, w16)
            n16 = -(plsc.load_gather(dis_v, [r16]) * weff
                    * plsc.load_gather(dis_v, [c16]))
            ci[u][pl.ds(g * L, L)] = c16
            for i in range(L):
                s = _splat(n16, i)
                j = g * L + i
                for cc in range(D // L):
                    xr[u][j, pl.ds(cc * L, L)] = (
                        xr[u][j, pl.ds(cc * L, L)] * s)
            return c
        lax.fori_loop(0, B // L, grp, 0)

    def issue_scatter(u):
        pltpu.async_copy(xr[u], y_sh.at[ci[u]], ss[u], add=True)

    def slot(u, b):
        wait_gather(u)
        scale(u)
        issue_scatter(u)

        @pl.when(b + 3 < NBT)
        def _():
            issue_pk(u, b + 3)
        v2 = (u + 2) % 3

        @pl.when(b + 2 < NBT)
        def _():
            @pl.when(b >= 1)
            def _():
                wait_scatter(v2)
            wait_pk(v2)
            issue_gather(v2)

    issue_pk(0, 0)
    issue_pk(1, 1)
    issue_pk(2, 2)
    wait_pk(0)
    issue_gather(0)
    wait_pk(1)
    issue_gather(1)

    def pipe_body(jj, c):
        b0 = 3 * jj
        slot(0, b0)
        slot(1, b0 + 1)
        slot(2, b0 + 2)
        return c
    lax.fori_loop(0, (NBT - 2) // 3, pipe_body, 0)
    slot(0, NBT - 2)
    slot(1, NBT - 1)
    wait_scatter(2)
    wait_scatter(0)
    wait_scatter(1)
    plsc.subcore_barrier()

    nlast = N - (NS - 1) * SLICE

    @pl.when(sid < NS - 1)
    def _():
        pltpu.sync_copy(y_sh.at[pl.ds(sid * SLICE, SLICE)],
                        y_h.at[cid, pl.ds(sid * SLICE, SLICE)])

    @pl.when(sid == NS - 1)
    def _():
        pltpu.sync_copy(y_sh.at[pl.ds((NS - 1) * SLICE, nlast)],
                        y_h.at[cid, pl.ds((NS - 1) * SLICE, nlast)])


def _dense_body(x_ref, y_ref, w0_ref, w1_ref, ci_ref, ct_ref, co_ref,
                wco_ref, wo_ref, bo_ref, o_ref):
    xs = x_ref[...]
    ys = y_ref[0] + y_ref[1]
    t = (jnp.dot(xs, w0_ref[...], preferred_element_type=jnp.float32)
         + jnp.dot(ys, w1_ref[...], preferred_element_type=jnp.float32))
    gi = jax.nn.sigmoid(t[:, 0:DH] + ci_ref[...])
    gt = jnp.tanh(t[:, DH:2 * DH] + ct_ref[...])
    cc = gi * gt
    go = jax.nn.sigmoid(t[:, 2 * DH:3 * DH] + co_ref[...]
                        + wco_ref[...] * cc)
    hh = go * jnp.tanh(cc)
    o_ref[...] = (jnp.dot(hh, wo_ref[...], preferred_element_type=jnp.float32)
                  + bo_ref[...])


_RB = 1000


def _dense(x, y2, w0c, w1c, ci, ct, co, wco, wo, bo):
    grid = (N // _RB,)
    return pl.pallas_call(
        _dense_body,
        grid=grid,
        in_specs=[
            pl.BlockSpec((_RB, D), lambda i: (i, 0)),
            pl.BlockSpec((NC, _RB, D), lambda i: (0, i, 0)),
            pl.BlockSpec((D, 3 * DH), lambda i: (0, 0)),
            pl.BlockSpec((D, 3 * DH), lambda i: (0, 0)),
            pl.BlockSpec((1, DH), lambda i: (0, 0)),
            pl.BlockSpec((1, DH), lambda i: (0, 0)),
            pl.BlockSpec((1, DH), lambda i: (0, 0)),
            pl.BlockSpec((1, DH), lambda i: (0, 0)),
            pl.BlockSpec((DH, 1), lambda i: (0, 0)),
            pl.BlockSpec((1, 1), lambda i: (0, 0)),
        ],
        out_specs=pl.BlockSpec((_RB, 1), lambda i: (i, 0)),
        out_shape=jax.ShapeDtypeStruct((N, 1), jnp.float32),
    )(x, y2, w0c, w1c, ci, ct, co, wco, wo, bo)


def kernel(x, edge_index, edge_weight, Wx0, Wx1, bx, Wh0, Wh1, bh, wc, b,
           W_out, b_out):
    row = edge_index[0]
    col = edge_index[1]

    parts = _deg_kernel(row, col, edge_weight)
    ewi = lax.bitcast_convert_type(edge_weight, jnp.int32)
    pk = jnp.stack([row.reshape(E // B, B), col.reshape(E // B, B),
                    ewi.reshape(E // B, B)], axis=1)
    y2 = _spmm_kernel(pk, parts, x)

    w0c = jnp.concatenate([Wx0[0], Wx0[2], Wx0[3]], axis=1)
    w1c = jnp.concatenate([Wx1[0], Wx1[2], Wx1[3]], axis=1)
    ci = (bx[0] + bh[0] + b[0]).reshape(1, DH)
    ct = (bx[2] + bh[2] + b[2]).reshape(1, DH)
    co = (bx[3] + bh[3] + b[3]).reshape(1, DH)
    wco = wc[2].reshape(1, DH)
    bo = b_out.reshape(1, 1)

    out = _dense(x, y2, w0c, w1c, ci, ct, co, wco, W_out, bo)
    return out[:, 0]
